# baseline (speedup 1.0000x reference)
import functools

import jax
import jax.numpy as jnp
from jax import lax
from jax.experimental import pallas as pl
from jax.experimental.pallas import tpu as pltpu
from jax.experimental.pallas import tpu_sc as plsc

WIN = 24
NH = 16
AREA = WIN * WIN
N = AREA + 1
NN = N * N
NREL = (2 * WIN - 1) * (2 * WIN - 1) + 3
B = 16

CH = 16384
NBLK = 21
NN_PAD = 22 * CH

NW = 32
PER_W = NN_PAD // NW
DCHUNK = 1408
NCHUNK = PER_W // DCHUNK
VPC = DCHUNK // 16


@functools.cache
def _make_sc_gather():
    mesh = plsc.VectorSubcoreMesh(core_axis_name="c", subcore_axis_name="s")

    @functools.partial(
        pl.kernel,
        mesh=mesh,
        out_type=jax.ShapeDtypeStruct((NH, NN_PAD), jnp.float32),
        scratch_types=[
            pltpu.VMEM((NREL * NH,), jnp.float32),
            pltpu.VMEM((PER_W,), jnp.int32),
            pltpu.VMEM((NH, DCHUNK), jnp.float32),
        ],
        compiler_params=pltpu.CompilerParams(needs_layout_passes=False),
    )
    def _sc_gather(table_hbm, idx_hbm, out_hbm, table_v, idx_v, buf_v):
        wid = lax.axis_index("s") * 2 + lax.axis_index("c")
        base = wid * PER_W
        pltpu.sync_copy(table_hbm, table_v)
        pltpu.sync_copy(idx_hbm.at[pl.ds(base, PER_W)], idx_v)

        for c in range(NCHUNK):
            def body(k, _):
                idx16 = idx_v[pl.ds(c * DCHUNK + k * 16, 16)]
                fidx = idx16 * NH
                for h in range(NH):
                    vals = plsc.load_gather(table_v, [fidx + h])
                    buf_v[h, pl.ds(k * 16, 16)] = vals
                return _
            lax.fori_loop(0, VPC, body, None)
            pltpu.sync_copy(
                buf_v, out_hbm.at[:, pl.ds(base + c * DCHUNK, DCHUNK)])

    return _sc_gather


def _add_body(attn_ref, bias_ref, out_ref):
    out_ref[...] = attn_ref[...] + bias_ref[...][None]


def kernel(attn, relative_position_bias_table, relative_position_index):
    idx = relative_position_index.reshape(-1).astype(jnp.int32)
    idx_pad = jnp.zeros((NN_PAD,), jnp.int32).at[:NN].set(idx)
    bias_t = _make_sc_gather()(relative_position_bias_table.reshape(-1),
                               idx_pad)

    attn_flat = attn.reshape(B, NH, NN)
    out = pl.pallas_call(
        _add_body,
        grid=(NBLK, B),
        in_specs=[
            pl.BlockSpec((1, NH, CH), lambda i, b: (b, 0, i)),
            pl.BlockSpec((NH, CH), lambda i, b: (0, i)),
        ],
        out_specs=pl.BlockSpec((1, NH, CH), lambda i, b: (b, 0, i)),
        out_shape=jax.ShapeDtypeStruct((B, NH, NN), jnp.float32),
    )(attn_flat, bias_t)
    return out.reshape(B, NH, N, N)


# speedup vs baseline: 2.1553x; 2.1553x over previous
import functools

import jax
import jax.numpy as jnp
from jax import lax
from jax.experimental import pallas as pl
from jax.experimental.pallas import tpu as pltpu
from jax.experimental.pallas import tpu_sc as plsc

WIN = 24
NH = 16
AREA = WIN * WIN
N = AREA + 1
NN = N * N
NREL = (2 * WIN - 1) * (2 * WIN - 1) + 3
B = 16

RB = 64
NRB = 10
NN_PAD = 360448

NW = 32
PER_W = NN_PAD // NW
DCHUNK = 1408
NCHUNK = PER_W // DCHUNK
VPC = DCHUNK // 16


@functools.cache
def _make_sc_gather():
    mesh = plsc.VectorSubcoreMesh(core_axis_name="c", subcore_axis_name="s")

    @functools.partial(
        pl.kernel,
        mesh=mesh,
        out_type=jax.ShapeDtypeStruct((NH, NN_PAD), jnp.float32),
        scratch_types=[
            pltpu.VMEM((NREL * NH,), jnp.float32),
            pltpu.VMEM((PER_W,), jnp.int32),
            pltpu.VMEM((NH, DCHUNK), jnp.float32),
        ],
        compiler_params=pltpu.CompilerParams(needs_layout_passes=False),
    )
    def _sc_gather(table_hbm, idx_hbm, out_hbm, table_v, idx_v, buf_v):
        wid = lax.axis_index("s") * 2 + lax.axis_index("c")
        base = wid * PER_W
        pltpu.sync_copy(table_hbm, table_v)
        pltpu.sync_copy(idx_hbm.at[pl.ds(base, PER_W)], idx_v)

        for c in range(NCHUNK):
            def body(k, _):
                idx16 = idx_v[pl.ds(c * DCHUNK + k * 16, 16)]
                fidx = idx16 * NH
                for h in range(NH):
                    vals = plsc.load_gather(table_v, [fidx + h])
                    buf_v[h, pl.ds(k * 16, 16)] = vals
                return _
            lax.fori_loop(0, VPC, body, None)
            pltpu.sync_copy(
                buf_v, out_hbm.at[:, pl.ds(base + c * DCHUNK, DCHUNK)])

    return _sc_gather


def _add_body(attn_ref, bias_ref, out_ref):
    out_ref[...] = attn_ref[...] + bias_ref[...][None]


def kernel(attn, relative_position_bias_table, relative_position_index):
    idx = relative_position_index.reshape(-1).astype(jnp.int32)
    idx_pad = jnp.zeros((NN_PAD,), jnp.int32).at[:NN].set(idx)
    bias_t = _make_sc_gather()(relative_position_bias_table.reshape(-1),
                               idx_pad)
    bias3 = bias_t[:, :NN].reshape(NH, N, N)

    out = pl.pallas_call(
        _add_body,
        grid=(NRB, B),
        in_specs=[
            pl.BlockSpec((1, NH, RB, N), lambda i, b: (b, 0, i, 0)),
            pl.BlockSpec((NH, RB, N), lambda i, b: (0, i, 0)),
        ],
        out_specs=pl.BlockSpec((1, NH, RB, N), lambda i, b: (b, 0, i, 0)),
        out_shape=jax.ShapeDtypeStruct((B, NH, N, N), jnp.float32),
    )(attn, bias3)
    return out


# speedup vs baseline: 2.2126x; 1.0266x over previous
import functools

import jax
import jax.numpy as jnp
from jax import lax
from jax.experimental import pallas as pl
from jax.experimental.pallas import tpu as pltpu
from jax.experimental.pallas import tpu_sc as plsc

WIN = 24
NH = 16
AREA = WIN * WIN
N = AREA + 1
NN = N * N
NREL = (2 * WIN - 1) * (2 * WIN - 1) + 3
B = 16

RB = 128
NRB = 5
NN_PAD = 360448

NW = 32
PER_W = NN_PAD // NW
DCHUNK = 1408
NCHUNK = PER_W // DCHUNK
VPC = DCHUNK // 16


@functools.cache
def _make_sc_gather():
    mesh = plsc.VectorSubcoreMesh(core_axis_name="c", subcore_axis_name="s")

    @functools.partial(
        pl.kernel,
        mesh=mesh,
        out_type=jax.ShapeDtypeStruct((NH, NN_PAD), jnp.float32),
        scratch_types=[
            pltpu.VMEM((NREL * NH,), jnp.float32),
            pltpu.VMEM((PER_W,), jnp.int32),
            pltpu.VMEM((NH, DCHUNK), jnp.float32),
        ],
        compiler_params=pltpu.CompilerParams(needs_layout_passes=False),
    )
    def _sc_gather(table_hbm, idx_hbm, out_hbm, table_v, idx_v, buf_v):
        wid = lax.axis_index("s") * 2 + lax.axis_index("c")
        base = wid * PER_W
        pltpu.sync_copy(table_hbm, table_v)
        pltpu.sync_copy(idx_hbm.at[pl.ds(base, PER_W)], idx_v)

        for c in range(NCHUNK):
            def body(k, _):
                idx16 = idx_v[pl.ds(c * DCHUNK + k * 16, 16)]
                fidx = idx16 * NH
                for h in range(NH):
                    vals = plsc.load_gather(table_v, [fidx + h])
                    buf_v[h, pl.ds(k * 16, 16)] = vals
                return _
            lax.fori_loop(0, VPC, body, None)
            pltpu.sync_copy(
                buf_v, out_hbm.at[:, pl.ds(base + c * DCHUNK, DCHUNK)])

    return _sc_gather


def _add_body(attn_ref, bias_ref, out_ref):
    out_ref[...] = attn_ref[...] + bias_ref[...][None]


def kernel(attn, relative_position_bias_table, relative_position_index):
    idx = relative_position_index.reshape(-1).astype(jnp.int32)
    idx_pad = jnp.zeros((NN_PAD,), jnp.int32).at[:NN].set(idx)
    bias_t = _make_sc_gather()(relative_position_bias_table.reshape(-1),
                               idx_pad)
    bias3 = bias_t[:, :NN].reshape(NH, N, N)

    out = pl.pallas_call(
        _add_body,
        grid=(NRB, B),
        in_specs=[
            pl.BlockSpec((1, NH, RB, N), lambda i, b: (b, 0, i, 0)),
            pl.BlockSpec((NH, RB, N), lambda i, b: (0, i, 0)),
        ],
        out_specs=pl.BlockSpec((1, NH, RB, N), lambda i, b: (b, 0, i, 0)),
        out_shape=jax.ShapeDtypeStruct((B, NH, N, N), jnp.float32),
    )(attn, bias3)
    return out
